# baseline (speedup 1.0000x reference)
import functools

import jax
import jax.numpy as jnp
from jax import lax
from jax.experimental import pallas as pl
from jax.experimental.pallas import tpu as pltpu, tpu_sc as plsc


@functools.cache
def _build(V, D, B):
    info = plsc.get_sparse_core_info()
    NC, NS, L = info.num_cores, info.num_subcores, info.num_lanes
    NW = NC * NS
    assert B % (8 * NW) == 0 and D % L == 0
    b_per_w = B // NW
    mesh = plsc.VectorSubcoreMesh(core_axis_name="c", subcore_axis_name="s")

    @functools.partial(
        pl.kernel,
        mesh=mesh,
        out_type=jax.ShapeDtypeStruct((D, B), jnp.float32),
        scratch_types=[
            pltpu.VMEM((b_per_w,), jnp.int32),
            pltpu.VMEM((b_per_w, 128), jnp.float32),
            pltpu.VMEM((D, b_per_w), jnp.float32),
            pltpu.SemaphoreType.DMA,
        ],
        compiler_params=pltpu.CompilerParams(
            skip_device_barrier=True, needs_layout_passes=False
        ),
    )
    def k(tp_hbm, idx_hbm, out_hbm, idx_v, buf_v, loc_v, sem):
        wid = lax.axis_index("s") * NC + lax.axis_index("c")
        base = wid * b_per_w
        pltpu.sync_copy(idx_hbm.at[pl.ds(base, b_per_w)], idx_v)
        pltpu.async_copy(tp_hbm.at[idx_v], buf_v, sem).wait()

        def sweep_g(g, _):
            rows = jnp.full((L,), g * L, jnp.int32) + lax.iota(jnp.int32, L)
            for j in range(D):
                vals = plsc.load_gather(buf_v, [rows, jnp.full((L,), j, jnp.int32)])
                loc_v[j, pl.ds(g * L, L)] = vals
            return 0

        lax.fori_loop(0, b_per_w // L, sweep_g, 0)
        pltpu.sync_copy(loc_v, out_hbm.at[:, pl.ds(base, b_per_w)])

    return k


def kernel(table, subject_ids):
    V, D = table.shape
    (B,) = subject_ids.shape
    tp = jnp.pad(table, ((0, 0), (0, 128 - D)))
    outT = _build(V, D, B)(tp, subject_ids.astype(jnp.int32))
    return outT.T


# speedup vs baseline: 1.5092x; 1.5092x over previous
import functools

import jax
import jax.numpy as jnp
from jax import lax
from jax.experimental import pallas as pl
from jax.experimental.pallas import tpu as pltpu, tpu_sc as plsc


@functools.cache
def _build(V, D, B):
    info = plsc.get_sparse_core_info()
    NC, NS, L = info.num_cores, info.num_subcores, info.num_lanes
    NW = NC * NS
    assert B % (8 * NW) == 0 and D % L == 0
    b_per_w = B // NW
    n_el = b_per_w * D
    mesh = plsc.VectorSubcoreMesh(core_axis_name="c", subcore_axis_name="s")

    @functools.partial(
        pl.kernel,
        mesh=mesh,
        out_type=jax.ShapeDtypeStruct((D, B), jnp.float32),
        scratch_types=[
            pltpu.VMEM((b_per_w,), jnp.int32),
            pltpu.VMEM((n_el,), jnp.int32),
            pltpu.VMEM((n_el,), jnp.float32),
            pltpu.SemaphoreType.DMA,
            pltpu.SemaphoreType.DMA,
        ],
        compiler_params=pltpu.CompilerParams(skip_device_barrier=True),
    )
    def k(t_hbm, idx_hbm, out_hbm, idx_v, gidx_v, buf_v, sem, sem2):
        wid = lax.axis_index("s") * NC + lax.axis_index("c")
        base = wid * b_per_w
        pltpu.sync_copy(idx_hbm.at[pl.ds(base, b_per_w)], idx_v)

        def fire_j(j, _):
            def build_g(g, _):
                v = idx_v[pl.ds(g * L, L)]
                gidx_v[pl.ds(j * b_per_w + g * L, L)] = v + j * V
                return 0

            lax.fori_loop(0, b_per_w // L, build_g, 0)
            pltpu.async_copy(
                t_hbm.at[gidx_v.at[pl.ds(j * b_per_w, b_per_w)]],
                buf_v.at[pl.ds(j * b_per_w, b_per_w)],
                sem,
            )
            return 0

        lax.fori_loop(0, D, fire_j, 0)

        def out_j(j, _):
            pltpu.make_async_copy(
                t_hbm.at[gidx_v.at[pl.ds(j * b_per_w, b_per_w)]],
                buf_v.at[pl.ds(j * b_per_w, b_per_w)],
                sem,
            ).wait()
            pltpu.async_copy(
                buf_v.at[pl.ds(j * b_per_w, b_per_w)],
                out_hbm.at[j, pl.ds(base, b_per_w)],
                sem2,
            )
            return 0

        lax.fori_loop(0, D, out_j, 0)

        def drain_j(j, _):
            pltpu.make_async_copy(
                buf_v.at[pl.ds(j * b_per_w, b_per_w)],
                out_hbm.at[j, pl.ds(base, b_per_w)],
                sem2,
            ).wait()
            return 0

        lax.fori_loop(0, D, drain_j, 0)

    return k


def kernel(table, subject_ids):
    V, D = table.shape
    (B,) = subject_ids.shape
    tflat = table.T.reshape(-1)
    outT = _build(V, D, B)(tflat, subject_ids.astype(jnp.int32))
    return outT.T
